# baseline (speedup 1.0000x reference)
import functools

import jax
import jax.numpy as jnp
from jax import lax
from jax.experimental import pallas as pl
from jax.experimental.pallas import tpu as pltpu
from jax.experimental.pallas import tpu_sc as plsc

B = 16384
V = 10
D = 512
NC = 2
NS = 16
NW = NC * NS
L = 16

B_SC = 2048
B_TC = B - B_SC
BPW = B_SC // NW
CH = 64
NCHUNK = BPW // CH

RB = 2048
BLK0 = B_SC // RB
NBLK = B_TC // RB
VP = 16

_mesh = plsc.VectorSubcoreMesh(
    core_axis_name="c", subcore_axis_name="s", num_cores=NC, num_subcores=NS
)


@functools.partial(
    pl.kernel,
    out_type=jax.ShapeDtypeStruct((B_SC, D), jnp.int32),
    mesh=_mesh,
    scratch_types=[
        pltpu.VMEM((NCHUNK, CH), jnp.int32),
        pltpu.VMEM((2, CH, D), jnp.int32),
        pltpu.SemaphoreType.DMA,
        pltpu.SemaphoreType.DMA,
        pltpu.SemaphoreType.DMA,
        pltpu.SemaphoreType.DMA,
    ],
)
def _gather_sc(tbl_hbm, idx_hbm, out_hbm, idx_v, rows_v, g0, g1, s0, s1):
    wid = lax.axis_index("s") * NC + lax.axis_index("c")
    base = wid * BPW
    gsem = (g0, g1)
    ssem = (s0, s1)

    for c in range(NCHUNK):
        pltpu.sync_copy(idx_hbm.at[pl.ds(base + c * CH, CH)], idx_v.at[c])

    off = wid * V
    for c in range(NCHUNK):
        for j in range(CH // L):
            sl = pl.ds(j * L, L)
            idx_v[c, sl] = idx_v[c, sl] + off

    def fire_gather(c):
        return pltpu.async_copy(
            tbl_hbm.at[idx_v.at[c]], rows_v.at[c % 2], gsem[c % 2]
        )

    def fire_store(c):
        return pltpu.async_copy(
            rows_v.at[c % 2], out_hbm.at[pl.ds(base + c * CH, CH)], ssem[c % 2]
        )

    gat = fire_gather(0)
    stores = {}
    for c in range(NCHUNK):
        if c + 1 < NCHUNK:
            if c - 1 >= 0:
                stores[c - 1].wait()
            nxt = fire_gather(c + 1)
        gat.wait()
        stores[c] = fire_store(c)
        if c + 1 < NCHUNK:
            gat = nxt
    for c in range(max(0, NCHUNK - 2), NCHUNK):
        stores[c].wait()


def _tc_body(idx_ref, tbl_ref, out_ref):
    idxb = idx_ref[0]
    oh = (jnp.broadcast_to(idxb, (VP, RB))
          == lax.broadcasted_iota(jnp.int32, (VP, RB), 0)).astype(jnp.float32)
    vals = lax.dot_general(
        oh, tbl_ref[...],
        dimension_numbers=(((0,), (0,)), ((), ())),
        preferred_element_type=jnp.float32,
    )
    out_ref[...] = vals.astype(jnp.int32)


_tc_fill = pl.pallas_call(
    _tc_body,
    grid=(NBLK,),
    in_specs=[
        pl.BlockSpec((1, 1, RB), lambda i: (BLK0 + i, 0, 0)),
        pl.BlockSpec((VP, D), lambda i: (0, 0)),
    ],
    out_specs=pl.BlockSpec((RB, D), lambda i: (BLK0 + i, 0)),
    out_shape=jax.ShapeDtypeStruct((B, D), jnp.int32),
)


def kernel(ind, mem):
    idx = ind.astype(jnp.int32)
    tbl_rep = jnp.broadcast_to(mem.astype(jnp.int32), (NW, V, D)).reshape(NW * V, D)
    sc_out = _gather_sc(tbl_rep, idx[:B_SC])
    idx3 = idx.reshape(B // RB, 1, RB)
    tblp = jnp.concatenate([mem, jnp.zeros((VP - V, D), jnp.float32)], axis=0)
    big = _tc_fill(idx3, tblp)
    return lax.dynamic_update_slice(big, sc_out, (0, 0))


# speedup vs baseline: 1.0311x; 1.0311x over previous
import functools

import jax
import jax.numpy as jnp
from jax import lax
from jax.experimental import pallas as pl
from jax.experimental.pallas import tpu as pltpu
from jax.experimental.pallas import tpu_sc as plsc

B = 16384
V = 10
D = 512
NC = 2
NS = 16
NW = NC * NS
L = 16

B_SC = 2048
B_TC = B - B_SC
BPW = B_SC // NW
CH = 64
NCHUNK = BPW // CH

RB = 2048
BLK0 = B_SC // RB
NBLK = B_TC // RB
VP = 16

_mesh = plsc.VectorSubcoreMesh(
    core_axis_name="c", subcore_axis_name="s", num_cores=NC, num_subcores=NS
)


@functools.partial(
    pl.kernel,
    out_type=jax.ShapeDtypeStruct((B_SC, D), jnp.int32),
    mesh=_mesh,
    scratch_types=[
        pltpu.VMEM((NCHUNK, CH), jnp.int32),
        pltpu.VMEM((2, CH, D), jnp.int32),
        pltpu.SemaphoreType.DMA,
        pltpu.SemaphoreType.DMA,
        pltpu.SemaphoreType.DMA,
        pltpu.SemaphoreType.DMA,
    ],
)
def _gather_sc(tbl_hbm, idx_hbm, out_hbm, idx_v, rows_v, g0, g1, s0, s1):
    wid = lax.axis_index("s") * NC + lax.axis_index("c")
    base = wid * BPW
    gsem = (g0, g1)
    ssem = (s0, s1)

    for c in range(NCHUNK):
        pltpu.sync_copy(idx_hbm.at[pl.ds(base + c * CH, CH)], idx_v.at[c])

    off = wid * V
    for c in range(NCHUNK):
        for j in range(CH // L):
            sl = pl.ds(j * L, L)
            idx_v[c, sl] = idx_v[c, sl] + off

    def fire_gather(c):
        return pltpu.async_copy(
            tbl_hbm.at[idx_v.at[c]], rows_v.at[c % 2], gsem[c % 2]
        )

    def fire_store(c):
        return pltpu.async_copy(
            rows_v.at[c % 2], out_hbm.at[pl.ds(base + c * CH, CH)], ssem[c % 2]
        )

    gat = fire_gather(0)
    stores = {}
    for c in range(NCHUNK):
        if c + 1 < NCHUNK:
            if c - 1 >= 0:
                stores[c - 1].wait()
            nxt = fire_gather(c + 1)
        gat.wait()
        stores[c] = fire_store(c)
        if c + 1 < NCHUNK:
            gat = nxt
    for c in range(max(0, NCHUNK - 2), NCHUNK):
        stores[c].wait()


def _tc_body(idx_ref, tbl_ref, out_ref):
    idxb = idx_ref[0]
    oh = (jnp.broadcast_to(idxb, (V, RB))
          == lax.broadcasted_iota(jnp.int32, (V, RB), 0)).astype(jnp.float32)
    vals = lax.dot_general(
        oh, tbl_ref[...],
        dimension_numbers=(((0,), (0,)), ((), ())),
        preferred_element_type=jnp.float32,
    )
    out_ref[...] = vals.astype(jnp.int32)


_tc_fill = pl.pallas_call(
    _tc_body,
    grid=(NBLK,),
    in_specs=[
        pl.BlockSpec((1, 1, RB), lambda i: (BLK0 + i, 0, 0)),
        pl.BlockSpec((V, D), lambda i: (0, 0)),
    ],
    out_specs=pl.BlockSpec((RB, D), lambda i: (BLK0 + i, 0)),
    out_shape=jax.ShapeDtypeStruct((B, D), jnp.int32),
)


def kernel(ind, mem):
    idx = ind.astype(jnp.int32)
    tbl_rep = jnp.broadcast_to(mem.astype(jnp.int32), (NW, V, D)).reshape(NW * V, D)
    sc_out = _gather_sc(tbl_rep, idx)
    idx3 = idx.reshape(B // RB, 1, RB)
    big = _tc_fill(idx3, mem)
    return lax.dynamic_update_slice(big, sc_out, (0, 0))
